# Initial kernel scaffold; baseline (speedup 1.0000x reference)
import jax
import jax.numpy as jnp
from jax.experimental import pallas as pl


def kernel(normal_features, extreme_features, edge_index, batch, gnn0_Wl, gnn0_bl, gnn0_Wr, gnn1_Wl, gnn1_bl, gnn1_Wr, ln_g_gamma, ln_g_beta, mlp_W1, mlp_b1, mlp_W2, mlp_b2, ln_m_gamma, ln_m_beta, projg_W, projg_b, projm_W, projm_b, gate_W, gate_b, fc_W, fc_b):
    raise NotImplementedError("write your pallas kernel here")


# speedup vs baseline: 2.5547x; 2.5547x over previous
import jax
import jax.numpy as jnp
from jax import lax
from jax.experimental import pallas as pl
from jax.experimental.pallas import tpu as pltpu
from jax.experimental.pallas import tpu_sc as plsc

_N = 10000
_E = 320000
_D = 128
_HID = 256
_G = 64
_CH = 128
_NTILE = 16
_RT = 160
_EPAD = _RT * _CH * _NTILE
_NP = 10112
_RPT = _NP // _NTILE
_DUMMY = _N

_R = 1000
_NBLK = _N // _R


def _make_sc_agg():
    mesh = plsc.VectorSubcoreMesh(core_axis_name="c", subcore_axis_name="s")
    outs = (jax.ShapeDtypeStruct((_NP, _D), jnp.float32),
            jax.ShapeDtypeStruct((_NP, _D), jnp.float32))
    scratch = (
        pltpu.VMEM((_CH,), jnp.int32),
        pltpu.VMEM((_CH,), jnp.int32),
        pltpu.VMEM((_CH, _D), jnp.float32),
        pltpu.VMEM_SHARED((_NP, _D), jnp.float32),
        pltpu.SemaphoreType.DMA,
    )

    def body(src_flat, dst_flat, tab_lo, tab_hi, zeros_np,
             agg_lo, agg_hi, src_v, dst_v, rows, acc_sh, sem):
        c = lax.axis_index("c")
        s = lax.axis_index("s")
        base = s * _RPT

        pltpu.sync_copy(zeros_np.at[pl.ds(base, _RPT)],
                        acc_sh.at[pl.ds(base, _RPT)])
        plsc.subcore_barrier()

        def edge_loop(tab):
            def chunk(i, _):
                ebase = (s * _RT + i) * _CH
                pltpu.sync_copy(src_flat.at[pl.ds(ebase, _CH)], src_v)
                pltpu.sync_copy(dst_flat.at[pl.ds(ebase, _CH)], dst_v)
                pltpu.async_copy(tab.at[src_v], rows, sem).wait()
                pltpu.sync_copy(rows, acc_sh.at[dst_v], add=True)
                return 0
            lax.fori_loop(0, _RT, chunk, 0)

        @pl.when(c == 0)
        def _():
            edge_loop(tab_lo)

        @pl.when(c == 1)
        def _():
            edge_loop(tab_hi)

        plsc.subcore_barrier()

        @pl.when(c == 0)
        def _():
            pltpu.sync_copy(acc_sh.at[pl.ds(base, _RPT)],
                            agg_lo.at[pl.ds(base, _RPT)])

        @pl.when(c == 1)
        def _():
            pltpu.sync_copy(acc_sh.at[pl.ds(base, _RPT)],
                            agg_hi.at[pl.ds(base, _RPT)])

    return pl.kernel(body, out_type=outs, mesh=mesh, scratch_types=scratch)


def _make_sc_count():
    mesh = plsc.VectorSubcoreMesh(core_axis_name="c", subcore_axis_name="s")
    outs = (jax.ShapeDtypeStruct((_NP, _D), jnp.float32),
            jax.ShapeDtypeStruct((_NP, _D), jnp.float32))
    scratch = (
        pltpu.VMEM((_CH,), jnp.int32),
        pltpu.VMEM((_CH, _D), jnp.float32),
        pltpu.VMEM_SHARED((_NP, _D), jnp.float32),
    )
    half = _RT // 2

    def body(dst_flat, zeros_np, ones_hbm, cnt0, cnt1,
             dst_v, ones_v, acc_sh):
        c = lax.axis_index("c")
        s = lax.axis_index("s")
        base = s * _RPT
        pltpu.sync_copy(zeros_np.at[pl.ds(base, _RPT)],
                        acc_sh.at[pl.ds(base, _RPT)])
        pltpu.sync_copy(ones_hbm, ones_v)
        plsc.subcore_barrier()

        def chunk(i, _):
            ebase = (s * _RT + c * half + i) * _CH
            pltpu.sync_copy(dst_flat.at[pl.ds(ebase, _CH)], dst_v)
            pltpu.sync_copy(ones_v, acc_sh.at[dst_v], add=True)
            return 0
        lax.fori_loop(0, half, chunk, 0)
        plsc.subcore_barrier()

        @pl.when(c == 0)
        def _():
            pltpu.sync_copy(acc_sh.at[pl.ds(base, _RPT)],
                            cnt0.at[pl.ds(base, _RPT)])

        @pl.when(c == 1)
        def _():
            pltpu.sync_copy(acc_sh.at[pl.ds(base, _RPT)],
                            cnt1.at[pl.ds(base, _RPT)])

    return pl.kernel(body, out_type=outs, mesh=mesh, scratch_types=scratch)


import functools as _functools


@_functools.lru_cache(maxsize=None)
def _sc_agg_fn():
    return _make_sc_agg()


@_functools.lru_cache(maxsize=None)
def _sc_count_fn():
    return _make_sc_count()


def _sc_agg(srcp, dstp, tlo, thi):
    zeros_np = jnp.zeros((_NP, _D), jnp.float32)
    return _sc_agg_fn()(srcp, dstp, tlo, thi, zeros_np)


def _sc_count(dstp):
    zeros_np = jnp.zeros((_NP, _D), jnp.float32)
    ones128 = jnp.ones((_CH, _D), jnp.float32)
    return _sc_count_fn()(dstp, zeros_np, ones128)


def _ln(x, g, b):
    mu = jnp.mean(x, axis=-1, keepdims=True)
    var = jnp.mean((x - mu) ** 2, axis=-1, keepdims=True)
    return (x - mu) / jnp.sqrt(var + 1e-5) * g + b


def _dot(a, b):
    return jnp.dot(a, b, preferred_element_type=jnp.float32)


def _stage2_body(nref, eref, alref, ahref, c0ref, c1ref, wl, blr, wr, w1,
                 b1r, w2, b2r, gmr, bmr, wpm, bpmr, hlo, hhi, hmp):
    x = jnp.concatenate([nref[...], eref[...]], axis=1)
    inv = 1.0 / jnp.maximum(c0ref[...][:, 0:1] + c1ref[...][:, 0:1], 1.0)
    mean = jnp.concatenate([alref[...], ahref[...]], axis=1) * inv
    h = jnp.maximum(_dot(mean, wl[...]) + blr[...] + _dot(x, wr[...]), 0.0)
    hlo[...] = h[:, :_D]
    hhi[...] = h[:, _D:]
    m = _dot(jnp.maximum(_dot(x, w1[...]) + b1r[...], 0.0), w2[...]) + b2r[...]
    m = _ln(m, gmr[...], bmr[...])
    hmp[...] = _dot(m, wpm[...]) + bpmr[...]


def _row_spec(w):
    return pl.BlockSpec((_R, w), lambda i: (i, 0))


def _full_spec(shape):
    return pl.BlockSpec(shape, lambda i: tuple(0 for _ in shape))


def _stage2(normal, extreme, agg_lo, agg_hi, cnt0, cnt1, wl, bl, wr, w1, b1,
            w2, b2, gm, bm, wpm, bpm):
    in_specs = [
        _row_spec(_D), _row_spec(_D), _row_spec(_D), _row_spec(_D),
        _row_spec(_D), _row_spec(_D),
        _full_spec((_HID, _HID)), _full_spec((1, _HID)),
        _full_spec((_HID, _HID)),
        _full_spec((_HID, _HID)), _full_spec((1, _HID)),
        _full_spec((_HID, _HID)), _full_spec((1, _HID)),
        _full_spec((1, _HID)), _full_spec((1, _HID)),
        _full_spec((_HID, _HID)), _full_spec((1, _HID)),
    ]
    out_specs = [_row_spec(_D), _row_spec(_D), _row_spec(_HID)]
    return pl.pallas_call(
        _stage2_body,
        grid=(_NBLK,),
        in_specs=in_specs,
        out_specs=out_specs,
        out_shape=[jax.ShapeDtypeStruct((_N, _D), jnp.float32),
                   jax.ShapeDtypeStruct((_N, _D), jnp.float32),
                   jax.ShapeDtypeStruct((_N, _HID), jnp.float32)],
    )(normal, extreme, agg_lo, agg_hi, cnt0, cnt1, wl, bl, wr, w1, b1, w2,
      b2, gm, bm, wpm, bpm)


def _stage4_body(alref, ahref, c0ref, c1ref, hloref, hhiref, hmpref, bref,
                 wl, blr, wr, ggr, gbr, wpg, bpgr, wg, bgr, fcw, fcbr, out,
                 gs_acc, gc_acc):
    i = pl.program_id(0)

    @pl.when(i == 0)
    def _():
        gs_acc[...] = jnp.zeros_like(gs_acc)
        gc_acc[...] = jnp.zeros_like(gc_acc)

    inv = 1.0 / jnp.maximum(c0ref[...][:, 0:1] + c1ref[...][:, 0:1], 1.0)
    mean = jnp.concatenate([alref[...], ahref[...]], axis=1) * inv
    hg0 = jnp.concatenate([hloref[...], hhiref[...]], axis=1)
    h = jnp.maximum(_dot(mean, wl[...]) + blr[...] + _dot(hg0, wr[...]), 0.0)
    h = _ln(h, ggr[...], gbr[...])
    hgp = _dot(h, wpg[...]) + bpgr[...]
    hmp = hmpref[...]
    wgm = wg[...]
    a = jax.nn.sigmoid(_dot(hgp, wgm[:_HID]) + _dot(hmp, wgm[_HID:]) + bgr[...])
    hf = a * hgp + (1.0 - a) * hmp

    oh = (bref[...] == lax.broadcasted_iota(jnp.int32, (1, _G), 1))
    oh = oh.astype(jnp.float32)
    dn = (((0,), (0,)), ((), ()))
    gs_acc[...] += lax.dot_general(oh, hf, dn,
                                   preferred_element_type=jnp.float32)
    gc_acc[...] += lax.dot_general(oh, jnp.ones((_R, 128), jnp.float32), dn,
                                   preferred_element_type=jnp.float32)

    @pl.when(i == _NBLK - 1)
    def _():
        g = gs_acc[...] / jnp.maximum(gc_acc[...][:, 0:1], 1.0)
        out[...] = jax.nn.sigmoid(_dot(g, fcw[...]) + fcbr[...])


def _stage4(agg_lo, agg_hi, cnt0, cnt1, hlo, hhi, hmp, batch2d, wl, bl, wr,
            gg, gb, wpg, bpg, wg, bg, fcw, fcb):
    in_specs = [
        _row_spec(_D), _row_spec(_D), _row_spec(_D), _row_spec(_D),
        _row_spec(_D), _row_spec(_D), _row_spec(_HID),
        _row_spec(1),
        _full_spec((_HID, _HID)), _full_spec((1, _HID)),
        _full_spec((_HID, _HID)),
        _full_spec((1, _HID)), _full_spec((1, _HID)),
        _full_spec((_HID, _HID)), _full_spec((1, _HID)),
        _full_spec((2 * _HID, _HID)), _full_spec((1, _HID)),
        _full_spec((_HID, 128)), _full_spec((1, 128)),
    ]
    return pl.pallas_call(
        _stage4_body,
        grid=(_NBLK,),
        in_specs=in_specs,
        out_specs=_full_spec((_G, 128)),
        out_shape=jax.ShapeDtypeStruct((_G, 128), jnp.float32),
        scratch_shapes=[pltpu.VMEM((_G, _HID), jnp.float32),
                        pltpu.VMEM((_G, 128), jnp.float32)],
    )(agg_lo, agg_hi, cnt0, cnt1, hlo, hhi, hmp, batch2d, wl, bl, wr, gg,
      gb, wpg, bpg, wg, bg, fcw, fcb)


def kernel(normal_features, extreme_features, edge_index, batch, gnn0_Wl,
           gnn0_bl, gnn0_Wr, gnn1_Wl, gnn1_bl, gnn1_Wr, ln_g_gamma,
           ln_g_beta, mlp_W1, mlp_b1, mlp_W2, mlp_b2, ln_m_gamma, ln_m_beta,
           projg_W, projg_b, projm_W, projm_b, gate_W, gate_b, fc_W, fc_b):
    pad = _EPAD - _E
    srcp = jnp.concatenate(
        [edge_index[0], jnp.zeros((pad,), edge_index.dtype)])
    dstp = jnp.concatenate(
        [edge_index[1], jnp.full((pad,), _DUMMY, edge_index.dtype)])

    r1 = lambda v: v.reshape(1, -1)

    cnt0, cnt1 = _sc_count(dstp)
    c0, c1 = cnt0[:_N], cnt1[:_N]
    agg0_lo, agg0_hi = _sc_agg(srcp, dstp, normal_features, extreme_features)
    hg_lo, hg_hi, hmp = _stage2(
        normal_features, extreme_features, agg0_lo[:_N], agg0_hi[:_N], c0, c1,
        gnn0_Wl, r1(gnn0_bl), gnn0_Wr, mlp_W1, r1(mlp_b1), mlp_W2, r1(mlp_b2),
        r1(ln_m_gamma), r1(ln_m_beta), projm_W, r1(projm_b))

    agg1_lo, agg1_hi = _sc_agg(srcp, dstp, hg_lo, hg_hi)

    fcw = jnp.pad(fc_W, ((0, 0), (0, 128 - fc_W.shape[1])))
    fcb = jnp.broadcast_to(fc_b.reshape(1, 1), (1, 128))
    outp = _stage4(
        agg1_lo[:_N], agg1_hi[:_N], c0, c1, hg_lo, hg_hi, hmp,
        batch.reshape(_N, 1), gnn1_Wl, r1(gnn1_bl), gnn1_Wr,
        r1(ln_g_gamma), r1(ln_g_beta), projg_W, r1(projg_b), gate_W,
        r1(gate_b), fcw, fcb)
    return outp[:, :1]


# speedup vs baseline: 3.2151x; 1.2585x over previous
import jax
import jax.numpy as jnp
from jax import lax
from jax.experimental import pallas as pl
from jax.experimental.pallas import tpu as pltpu
from jax.experimental.pallas import tpu_sc as plsc

_N = 10000
_E = 320000
_D = 128
_HID = 256
_G = 64
_CH = 128
_NTILE = 16
_RT = 160
_EPAD = _RT * _CH * _NTILE
_NP = 10112
_RPT = _NP // _NTILE
_DUMMY = _N

_R = 1000
_NBLK = _N // _R


_GRP = 8


def _make_sc_agg():
    mesh = plsc.VectorSubcoreMesh(core_axis_name="c", subcore_axis_name="s")
    outs = (jax.ShapeDtypeStruct((_NP, _D), jnp.float32),
            jax.ShapeDtypeStruct((_NP, _D), jnp.float32))
    scratch = (
        pltpu.VMEM((_GRP, _CH), jnp.int32),
        pltpu.VMEM((_GRP, _CH), jnp.int32),
        pltpu.VMEM((_CH, _D), jnp.float32),
        pltpu.VMEM((_CH, _D), jnp.float32),
        pltpu.VMEM_SHARED((_NP, _D), jnp.float32),
        pltpu.SemaphoreType.DMA,
        pltpu.SemaphoreType.DMA,
        pltpu.SemaphoreType.DMA,
        pltpu.SemaphoreType.DMA,
    )

    def body(src2d, dst2d, tab_lo, tab_hi, zeros_np,
             agg_lo, agg_hi, src_v, dst_v, rowsA, rowsB, acc_sh,
             gsA, gsB, ssA, ssB):
        c = lax.axis_index("c")
        s = lax.axis_index("s")
        base = s * _RPT

        pltpu.sync_copy(zeros_np.at[pl.ds(base, _RPT)],
                        acc_sh.at[pl.ds(base, _RPT)])
        plsc.subcore_barrier()

        bufs = (rowsA, rowsB)
        gsems = (gsA, gsB)
        ssems = (ssA, ssB)

        def edge_loop(tab):
            def group(g, _):
                rbase = s * _RT + g * _GRP
                pltpu.sync_copy(src2d.at[pl.ds(rbase, _GRP)], src_v)
                pltpu.sync_copy(dst2d.at[pl.ds(rbase, _GRP)], dst_v)
                gd = [None, None]
                gd[0] = pltpu.async_copy(tab.at[src_v.at[0]], bufs[0],
                                         gsems[0])
                gd[1] = pltpu.async_copy(tab.at[src_v.at[1]], bufs[1],
                                         gsems[1])
                for r in range(_GRP):
                    p = r % 2
                    gd[p].wait()
                    sd = pltpu.async_copy(bufs[p], acc_sh.at[dst_v.at[r]],
                                          ssems[p], add=True)
                    sd.wait()
                    if r + 2 < _GRP:
                        gd[p] = pltpu.async_copy(tab.at[src_v.at[r + 2]],
                                                 bufs[p], gsems[p])
                return 0
            lax.fori_loop(0, _RT // _GRP, group, 0)

        @pl.when(c == 0)
        def _():
            edge_loop(tab_lo)

        @pl.when(c == 1)
        def _():
            edge_loop(tab_hi)

        plsc.subcore_barrier()

        @pl.when(c == 0)
        def _():
            pltpu.sync_copy(acc_sh.at[pl.ds(base, _RPT)],
                            agg_lo.at[pl.ds(base, _RPT)])

        @pl.when(c == 1)
        def _():
            pltpu.sync_copy(acc_sh.at[pl.ds(base, _RPT)],
                            agg_hi.at[pl.ds(base, _RPT)])

    return pl.kernel(body, out_type=outs, mesh=mesh, scratch_types=scratch)


def _make_sc_count():
    mesh = plsc.VectorSubcoreMesh(core_axis_name="c", subcore_axis_name="s")
    outs = (jax.ShapeDtypeStruct((_NP, _D), jnp.float32),
            jax.ShapeDtypeStruct((_NP, _D), jnp.float32))
    scratch = (
        pltpu.VMEM((_CH,), jnp.int32),
        pltpu.VMEM((_CH, _D), jnp.float32),
        pltpu.VMEM_SHARED((_NP, _D), jnp.float32),
    )
    half = _RT // 2

    def body(dst_flat, zeros_np, ones_hbm, cnt0, cnt1,
             dst_v, ones_v, acc_sh):
        c = lax.axis_index("c")
        s = lax.axis_index("s")
        base = s * _RPT
        pltpu.sync_copy(zeros_np.at[pl.ds(base, _RPT)],
                        acc_sh.at[pl.ds(base, _RPT)])
        pltpu.sync_copy(ones_hbm, ones_v)
        plsc.subcore_barrier()

        def chunk(i, _):
            ebase = (s * _RT + c * half + i) * _CH
            pltpu.sync_copy(dst_flat.at[pl.ds(ebase, _CH)], dst_v)
            pltpu.sync_copy(ones_v, acc_sh.at[dst_v], add=True)
            return 0
        lax.fori_loop(0, half, chunk, 0)
        plsc.subcore_barrier()

        @pl.when(c == 0)
        def _():
            pltpu.sync_copy(acc_sh.at[pl.ds(base, _RPT)],
                            cnt0.at[pl.ds(base, _RPT)])

        @pl.when(c == 1)
        def _():
            pltpu.sync_copy(acc_sh.at[pl.ds(base, _RPT)],
                            cnt1.at[pl.ds(base, _RPT)])

    return pl.kernel(body, out_type=outs, mesh=mesh, scratch_types=scratch)


import functools as _functools


@_functools.lru_cache(maxsize=None)
def _sc_agg_fn():
    return _make_sc_agg()


@_functools.lru_cache(maxsize=None)
def _sc_count_fn():
    return _make_sc_count()


def _sc_agg(srcp, dstp, tlo, thi):
    zeros_np = jnp.zeros((_NP, _D), jnp.float32)
    return _sc_agg_fn()(srcp, dstp, tlo, thi, zeros_np)


def _sc_count(dstp):
    zeros_np = jnp.zeros((_NP, _D), jnp.float32)
    ones128 = jnp.ones((_CH, _D), jnp.float32)
    return _sc_count_fn()(dstp, zeros_np, ones128)


def _ln(x, g, b):
    mu = jnp.mean(x, axis=-1, keepdims=True)
    var = jnp.mean((x - mu) ** 2, axis=-1, keepdims=True)
    return (x - mu) / jnp.sqrt(var + 1e-5) * g + b


def _dot(a, b):
    return jnp.dot(a, b, preferred_element_type=jnp.float32)


def _stage2_body(nref, eref, alref, ahref, c0ref, c1ref, wl, blr, wr, w1,
                 b1r, w2, b2r, gmr, bmr, wpm, bpmr, hlo, hhi, hmp):
    x = jnp.concatenate([nref[...], eref[...]], axis=1)
    inv = 1.0 / jnp.maximum(c0ref[...][:, 0:1] + c1ref[...][:, 0:1], 1.0)
    mean = jnp.concatenate([alref[...], ahref[...]], axis=1) * inv
    h = jnp.maximum(_dot(mean, wl[...]) + blr[...] + _dot(x, wr[...]), 0.0)
    hlo[...] = h[:, :_D]
    hhi[...] = h[:, _D:]
    m = _dot(jnp.maximum(_dot(x, w1[...]) + b1r[...], 0.0), w2[...]) + b2r[...]
    m = _ln(m, gmr[...], bmr[...])
    hmp[...] = _dot(m, wpm[...]) + bpmr[...]


def _row_spec(w):
    return pl.BlockSpec((_R, w), lambda i: (i, 0))


def _full_spec(shape):
    return pl.BlockSpec(shape, lambda i: tuple(0 for _ in shape))


def _stage2(normal, extreme, agg_lo, agg_hi, cnt0, cnt1, wl, bl, wr, w1, b1,
            w2, b2, gm, bm, wpm, bpm):
    in_specs = [
        _row_spec(_D), _row_spec(_D), _row_spec(_D), _row_spec(_D),
        _row_spec(_D), _row_spec(_D),
        _full_spec((_HID, _HID)), _full_spec((1, _HID)),
        _full_spec((_HID, _HID)),
        _full_spec((_HID, _HID)), _full_spec((1, _HID)),
        _full_spec((_HID, _HID)), _full_spec((1, _HID)),
        _full_spec((1, _HID)), _full_spec((1, _HID)),
        _full_spec((_HID, _HID)), _full_spec((1, _HID)),
    ]
    out_specs = [_row_spec(_D), _row_spec(_D), _row_spec(_HID)]
    return pl.pallas_call(
        _stage2_body,
        grid=(_NBLK,),
        in_specs=in_specs,
        out_specs=out_specs,
        out_shape=[jax.ShapeDtypeStruct((_N, _D), jnp.float32),
                   jax.ShapeDtypeStruct((_N, _D), jnp.float32),
                   jax.ShapeDtypeStruct((_N, _HID), jnp.float32)],
    )(normal, extreme, agg_lo, agg_hi, cnt0, cnt1, wl, bl, wr, w1, b1, w2,
      b2, gm, bm, wpm, bpm)


def _stage4_body(alref, ahref, c0ref, c1ref, hloref, hhiref, hmpref, bref,
                 wl, blr, wr, ggr, gbr, wpg, bpgr, wg, bgr, fcw, fcbr, out,
                 gs_acc, gc_acc):
    i = pl.program_id(0)

    @pl.when(i == 0)
    def _():
        gs_acc[...] = jnp.zeros_like(gs_acc)
        gc_acc[...] = jnp.zeros_like(gc_acc)

    inv = 1.0 / jnp.maximum(c0ref[...][:, 0:1] + c1ref[...][:, 0:1], 1.0)
    mean = jnp.concatenate([alref[...], ahref[...]], axis=1) * inv
    hg0 = jnp.concatenate([hloref[...], hhiref[...]], axis=1)
    h = jnp.maximum(_dot(mean, wl[...]) + blr[...] + _dot(hg0, wr[...]), 0.0)
    h = _ln(h, ggr[...], gbr[...])
    hgp = _dot(h, wpg[...]) + bpgr[...]
    hmp = hmpref[...]
    wgm = wg[...]
    a = jax.nn.sigmoid(_dot(hgp, wgm[:_HID]) + _dot(hmp, wgm[_HID:]) + bgr[...])
    hf = a * hgp + (1.0 - a) * hmp

    oh = (bref[...] == lax.broadcasted_iota(jnp.int32, (1, _G), 1))
    oh = oh.astype(jnp.float32)
    dn = (((0,), (0,)), ((), ()))
    gs_acc[...] += lax.dot_general(oh, hf, dn,
                                   preferred_element_type=jnp.float32)
    gc_acc[...] += lax.dot_general(oh, jnp.ones((_R, 128), jnp.float32), dn,
                                   preferred_element_type=jnp.float32)

    @pl.when(i == _NBLK - 1)
    def _():
        g = gs_acc[...] / jnp.maximum(gc_acc[...][:, 0:1], 1.0)
        out[...] = jax.nn.sigmoid(_dot(g, fcw[...]) + fcbr[...])


def _stage4(agg_lo, agg_hi, cnt0, cnt1, hlo, hhi, hmp, batch2d, wl, bl, wr,
            gg, gb, wpg, bpg, wg, bg, fcw, fcb):
    in_specs = [
        _row_spec(_D), _row_spec(_D), _row_spec(_D), _row_spec(_D),
        _row_spec(_D), _row_spec(_D), _row_spec(_HID),
        _row_spec(1),
        _full_spec((_HID, _HID)), _full_spec((1, _HID)),
        _full_spec((_HID, _HID)),
        _full_spec((1, _HID)), _full_spec((1, _HID)),
        _full_spec((_HID, _HID)), _full_spec((1, _HID)),
        _full_spec((2 * _HID, _HID)), _full_spec((1, _HID)),
        _full_spec((_HID, 128)), _full_spec((1, 128)),
    ]
    return pl.pallas_call(
        _stage4_body,
        grid=(_NBLK,),
        in_specs=in_specs,
        out_specs=_full_spec((_G, 128)),
        out_shape=jax.ShapeDtypeStruct((_G, 128), jnp.float32),
        scratch_shapes=[pltpu.VMEM((_G, _HID), jnp.float32),
                        pltpu.VMEM((_G, 128), jnp.float32)],
    )(agg_lo, agg_hi, cnt0, cnt1, hlo, hhi, hmp, batch2d, wl, bl, wr, gg,
      gb, wpg, bpg, wg, bg, fcw, fcb)


def kernel(normal_features, extreme_features, edge_index, batch, gnn0_Wl,
           gnn0_bl, gnn0_Wr, gnn1_Wl, gnn1_bl, gnn1_Wr, ln_g_gamma,
           ln_g_beta, mlp_W1, mlp_b1, mlp_W2, mlp_b2, ln_m_gamma, ln_m_beta,
           projg_W, projg_b, projm_W, projm_b, gate_W, gate_b, fc_W, fc_b):
    pad = _EPAD - _E
    srcp = jnp.concatenate(
        [edge_index[0], jnp.zeros((pad,), edge_index.dtype)])
    dstp = jnp.concatenate(
        [edge_index[1], jnp.full((pad,), _DUMMY, edge_index.dtype)])
    src2d = srcp.reshape(_EPAD // _CH, _CH)
    dst2d = dstp.reshape(_EPAD // _CH, _CH)

    r1 = lambda v: v.reshape(1, -1)

    cnt0, cnt1 = _sc_count(dstp)
    c0, c1 = cnt0[:_N], cnt1[:_N]
    agg0_lo, agg0_hi = _sc_agg(src2d, dst2d, normal_features,
                               extreme_features)
    hg_lo, hg_hi, hmp = _stage2(
        normal_features, extreme_features, agg0_lo[:_N], agg0_hi[:_N], c0, c1,
        gnn0_Wl, r1(gnn0_bl), gnn0_Wr, mlp_W1, r1(mlp_b1), mlp_W2, r1(mlp_b2),
        r1(ln_m_gamma), r1(ln_m_beta), projm_W, r1(projm_b))

    agg1_lo, agg1_hi = _sc_agg(src2d, dst2d, hg_lo, hg_hi)

    fcw = jnp.pad(fc_W, ((0, 0), (0, 128 - fc_W.shape[1])))
    fcb = jnp.broadcast_to(fc_b.reshape(1, 1), (1, 128))
    outp = _stage4(
        agg1_lo[:_N], agg1_hi[:_N], c0, c1, hg_lo, hg_hi, hmp,
        batch.reshape(_N, 1), gnn1_Wl, r1(gnn1_bl), gnn1_Wr,
        r1(ln_g_gamma), r1(ln_g_beta), projg_W, r1(projg_b), gate_W,
        r1(gate_b), fcw, fcb)
    return outp[:, :1]
